# Initial kernel scaffold; baseline (speedup 1.0000x reference)
import functools

import jax
import jax.numpy as jnp
from jax import lax
from jax.experimental import pallas as pl
from jax.experimental.pallas import tpu as pltpu
from jax.experimental.pallas import tpu_sc as plsc

_N = 100000
_NN = _N + 1
_B = 32
_E = 1600000
_NC = 2
_NS = 16
_L = 16

_EPAD = 1638400
_ROWS = _EPAD // 128
_RPW = _ROWS // _NS
_CHUNKS = _RPW // 8

_WPN = 6256
_NP = _WPN * _NS

_mesh = plsc.VectorSubcoreMesh(core_axis_name="c", subcore_axis_name="s")


@jax.jit
def _circuit_sc(xtr, src2, des2, par2):
    @functools.partial(
        pl.kernel,
        out_type=jax.ShapeDtypeStruct((_NC, _NP, _L), jnp.float32),
        mesh=_mesh,
        scratch_types=[
            pltpu.VMEM((8, 128), jnp.int32),
            pltpu.VMEM((8, 128), jnp.int32),
            pltpu.VMEM((8, 128), jnp.float32),
            pltpu.VMEM((8, 128), jnp.int32),
            pltpu.VMEM((8, 128), jnp.int32),
            pltpu.VMEM((1024, _L), jnp.float32),
            pltpu.VMEM((1024, _L), jnp.float32),
            pltpu.VMEM((1024, _L), jnp.float32),
            pltpu.VMEM((1024, _L), jnp.float32),
            pltpu.VMEM_SHARED((_NP, _L), jnp.float32),
        ],
    )
    def k(xtr_hbm, src_hbm, des_hbm, par_hbm, out_hbm,
          srcb, desb, parb, gsrcb, gdesb, vsb, vdb, iposb, inegb, acc):
        c = lax.axis_index("c")
        s = lax.axis_index("s")

        @pl.loop(0, 1024, unroll=8)
        def _(i):
            vsb[i] = jnp.zeros((_L,), jnp.float32)

        zbase = s * _WPN
        for tk in range(_WPN // 1024):
            pltpu.sync_copy(vsb.at[pl.ds(0, 1024)],
                            acc.at[pl.ds(zbase + tk * 1024, 1024)])
        _rem = _WPN % 1024
        if _rem:
            pltpu.sync_copy(vsb.at[pl.ds(0, _rem)],
                            acc.at[pl.ds(zbase + (_WPN // 1024) * 1024, _rem)])
        plsc.subcore_barrier()

        cvec = jnp.zeros((_L,), jnp.int32) + c
        row_base = s * _RPW

        @pl.loop(0, _CHUNKS)
        def _(chunk):
            off = row_base + chunk * 8
            pltpu.sync_copy(src_hbm.at[pl.ds(off, 8)], srcb)
            pltpu.sync_copy(des_hbm.at[pl.ds(off, 8)], desb)
            pltpu.sync_copy(par_hbm.at[pl.ds(off, 8)], parb)

            for r in range(8):
                @pl.loop(0, 8)
                def _(g, r=r):
                    v = srcb[r, pl.ds(g * _L, _L)]
                    gsrcb[r, pl.ds(g * _L, _L)] = v + v + cvec
                    w = desb[r, pl.ds(g * _L, _L)]
                    gdesb[r, pl.ds(g * _L, _L)] = w + w + cvec

            for j in range(8):
                pltpu.sync_copy(xtr_hbm.at[gsrcb.at[j]],
                                vsb.at[pl.ds(j * 128, 128)])
                pltpu.sync_copy(xtr_hbm.at[gdesb.at[j]],
                                vdb.at[pl.ds(j * 128, 128)])

            for r in range(8):
                @plsc.parallel_loop(0, 128, 1, unroll=8)
                def _(e, r=r):
                    p = parb[r, e]
                    row = p * (vsb[r * 128 + e] - vdb[r * 128 + e])
                    iposb[r * 128 + e] = row
                    inegb[r * 128 + e] = -row

            for j in range(8):
                pltpu.sync_copy(inegb.at[pl.ds(j * 128, 128)],
                                acc.at[srcb.at[j]], add=True)
                pltpu.sync_copy(iposb.at[pl.ds(j * 128, 128)],
                                acc.at[desb.at[j]], add=True)

        plsc.subcore_barrier()

        wbase = s * _WPN
        for tk in range(_WPN // 1024):
            pltpu.sync_copy(acc.at[pl.ds(wbase + tk * 1024, 1024)],
                            out_hbm.at[c, pl.ds(wbase + tk * 1024, 1024)])
        if _rem:
            pltpu.sync_copy(acc.at[pl.ds(wbase + (_WPN // 1024) * 1024, _rem)],
                            out_hbm.at[c, pl.ds(wbase + (_WPN // 1024) * 1024, _rem)])

    return k(xtr, src2, des2, par2)


def kernel(t, x, src, des, param):
    del t
    aux_t = jnp.concatenate([jnp.zeros((1, _B), x.dtype), x.T], axis=0)
    xtr = aux_t.reshape(_NC * _NN, _L)

    pad = _EPAD - _E
    src2 = jnp.concatenate([src, jnp.zeros((pad,), jnp.int32)]).reshape(_ROWS, 128)
    des2 = jnp.concatenate([des, jnp.zeros((pad,), jnp.int32)]).reshape(_ROWS, 128)
    par2 = jnp.concatenate([param, jnp.zeros((pad,), param.dtype)]).reshape(_ROWS, 128)

    out = _circuit_sc(xtr, src2, des2, par2)
    res = jnp.concatenate([out[0, 1:_NN, :], out[1, 1:_NN, :]], axis=-1)
    return res.T


# speedup vs baseline: 10.2446x; 10.2446x over previous
import functools

import jax
import jax.numpy as jnp
from jax import lax
from jax.experimental import pallas as pl
from jax.experimental.pallas import tpu as pltpu
from jax.experimental.pallas import tpu_sc as plsc

_N = 100000
_NN = _N + 1
_B = 32
_E = 1600000
_NC = 2
_NS = 16
_L = 16

_EPAD = 1638400
_ROWS = _EPAD // 128
_RPW = _ROWS // _NS
_CR = 2
_CHUNKS = _RPW // _CR
_CE = _CR * 128

_WPN = 6256
_NP = _WPN * _NS

_mesh = plsc.VectorSubcoreMesh(core_axis_name="c", subcore_axis_name="s")

_set = lambda: [
    pltpu.VMEM((_CR, 128), jnp.int32),
    pltpu.VMEM((_CR, 128), jnp.int32),
    pltpu.VMEM((_CR, 128), jnp.float32),
    pltpu.VMEM((_CR, 128), jnp.int32),
    pltpu.VMEM((_CR, 128), jnp.int32),
    pltpu.VMEM((_CE, _L), jnp.float32),
    pltpu.VMEM((_CE, _L), jnp.float32),
]


@jax.jit
def _circuit_sc(xtr, src2, des2, par2):
    @functools.partial(
        pl.kernel,
        out_type=jax.ShapeDtypeStruct((_NC, _NP, _L), jnp.float32),
        mesh=_mesh,
        scratch_types=_set() + _set() + [
            pltpu.VMEM_SHARED((_NP, _L), jnp.float32),
            pltpu.SemaphoreType.DMA,
            pltpu.SemaphoreType.DMA,
            pltpu.SemaphoreType.DMA,
            pltpu.SemaphoreType.DMA,
            pltpu.SemaphoreType.DMA,
            pltpu.SemaphoreType.DMA,
        ],
        compiler_params=pltpu.CompilerParams(use_tc_tiling_on_sc=False),
    )
    def k(xtr_hbm, src_hbm, des_hbm, par_hbm, out_hbm,
          src0, des0, par0, gs0, gd0, vs0, vd0,
          src1, des1, par1, gs1, gd1, vs1, vd1,
          acc, lsem0, lsem1, gsem0, gsem1, ssem0, ssem1):
        c = lax.axis_index("c")
        s = lax.axis_index("s")
        sets = ((src0, des0, par0, gs0, gd0, vs0, vd0, lsem0, gsem0, ssem0),
                (src1, des1, par1, gs1, gd1, vs1, vd1, lsem1, gsem1, ssem1))

        @pl.loop(0, _CE, unroll=8)
        def _(i):
            vs0[i] = jnp.zeros((_L,), jnp.float32)

        zbase = s * _WPN
        for tk in range(_WPN // _CE):
            pltpu.sync_copy(vs0.at[pl.ds(0, _CE)],
                            acc.at[pl.ds(zbase + tk * _CE, _CE)])
        _rem = _WPN % _CE
        if _rem:
            pltpu.sync_copy(vs0.at[pl.ds(0, _rem)],
                            acc.at[pl.ds(zbase + (_WPN // _CE) * _CE, _rem)])
        plsc.subcore_barrier()

        cvec = jnp.zeros((_L,), jnp.int32) + c
        row_base = s * _RPW

        def fire_loads(chunk, p):
            (srcb, desb, parb, _, _, _, _, lsem, _, _) = sets[p]
            off = row_base + chunk * _CR
            pltpu.async_copy(src_hbm.at[pl.ds(off, _CR)], srcb, lsem)
            pltpu.async_copy(des_hbm.at[pl.ds(off, _CR)], desb, lsem)
            pltpu.async_copy(par_hbm.at[pl.ds(off, _CR)], parb, lsem)

        def drain_loads(p):
            (srcb, desb, parb, _, _, _, _, lsem, _, _) = sets[p]
            pltpu.make_async_copy(src_hbm.at[pl.ds(0, _CR)], srcb, lsem).wait()
            pltpu.make_async_copy(des_hbm.at[pl.ds(0, _CR)], desb, lsem).wait()
            pltpu.make_async_copy(par_hbm.at[pl.ds(0, _CR)], parb, lsem).wait()

        def fire_gathers(p):
            (srcb, desb, _, gsb, gdb, vsb, vdb, _, gsem, _) = sets[p]
            for r in range(_CR):
                @pl.loop(0, 8)
                def _(g, r=r):
                    v = srcb[r, pl.ds(g * _L, _L)]
                    gsb[r, pl.ds(g * _L, _L)] = v + v + cvec
                    w = desb[r, pl.ds(g * _L, _L)]
                    gdb[r, pl.ds(g * _L, _L)] = w + w + cvec
            for j in range(_CR):
                pltpu.async_copy(xtr_hbm.at[gsb.at[j]],
                                 vsb.at[pl.ds(j * 128, 128)], gsem)
                pltpu.async_copy(xtr_hbm.at[gdb.at[j]],
                                 vdb.at[pl.ds(j * 128, 128)], gsem)

        def drain_gathers(p):
            (_, _, _, gsb, gdb, vsb, vdb, _, gsem, _) = sets[p]
            for j in range(_CR):
                pltpu.make_async_copy(xtr_hbm.at[gsb.at[j]],
                                      vsb.at[pl.ds(j * 128, 128)], gsem).wait()
                pltpu.make_async_copy(xtr_hbm.at[gdb.at[j]],
                                      vdb.at[pl.ds(j * 128, 128)], gsem).wait()

        def compute_and_fire_scatters(p):
            (srcb, desb, parb, _, _, vsb, vdb, _, _, ssem) = sets[p]
            for r in range(_CR):
                @plsc.parallel_loop(0, 128, 16)
                def _(e0, r=r):
                    pv = parb[r, pl.ds(e0, _L)]
                    for i in range(_L):
                        e = r * 128 + e0 + i
                        row = pv[i] * (vsb[e] - vdb[e])
                        vsb[e] = row
                        vdb[e] = -row
            for j in range(_CR):
                pltpu.async_copy(vdb.at[pl.ds(j * 128, 128)],
                                 acc.at[srcb.at[j]], ssem, add=True)
                pltpu.async_copy(vsb.at[pl.ds(j * 128, 128)],
                                 acc.at[desb.at[j]], ssem, add=True)

        def drain_scatters(p):
            (srcb, desb, _, _, _, vsb, vdb, _, _, ssem) = sets[p]
            for j in range(_CR):
                pltpu.make_async_copy(vdb.at[pl.ds(j * 128, 128)],
                                      acc.at[srcb.at[j]], ssem).wait()
                pltpu.make_async_copy(vsb.at[pl.ds(j * 128, 128)],
                                      acc.at[desb.at[j]], ssem).wait()

        fire_loads(0, 0)
        fire_loads(1, 1)
        drain_loads(0)
        fire_gathers(0)

        @pl.loop(0, _CHUNKS - 2, step=2)
        def _(chunk):
            drain_gathers(0)
            compute_and_fire_scatters(0)
            drain_loads(1)
            fire_gathers(1)
            drain_scatters(0)
            fire_loads(chunk + 2, 0)
            drain_gathers(1)
            compute_and_fire_scatters(1)
            drain_loads(0)
            fire_gathers(0)
            drain_scatters(1)
            fire_loads(chunk + 3, 1)

        drain_gathers(0)
        compute_and_fire_scatters(0)
        drain_loads(1)
        fire_gathers(1)
        drain_scatters(0)
        drain_gathers(1)
        compute_and_fire_scatters(1)
        drain_scatters(1)

        plsc.subcore_barrier()

        wbase = s * _WPN
        for tk in range(_WPN // _CE):
            pltpu.sync_copy(acc.at[pl.ds(wbase + tk * _CE, _CE)],
                            out_hbm.at[c, pl.ds(wbase + tk * _CE, _CE)])
        if _rem:
            pltpu.sync_copy(acc.at[pl.ds(wbase + (_WPN // _CE) * _CE, _rem)],
                            out_hbm.at[c, pl.ds(wbase + (_WPN // _CE) * _CE, _rem)])

    return k(xtr, src2, des2, par2)


def kernel(t, x, src, des, param):
    del t
    aux_t = jnp.concatenate([jnp.zeros((1, _B), x.dtype), x.T], axis=0)
    xtr = aux_t.reshape(_NC * _NN, _L)

    pad = _EPAD - _E
    src2 = jnp.concatenate([src, jnp.zeros((pad,), jnp.int32)]).reshape(_ROWS, 128)
    des2 = jnp.concatenate([des, jnp.zeros((pad,), jnp.int32)]).reshape(_ROWS, 128)
    par2 = jnp.concatenate([param, jnp.zeros((pad,), param.dtype)]).reshape(_ROWS, 128)

    out = _circuit_sc(xtr, src2, des2, par2)
    res = jnp.concatenate([out[0, 1:_NN, :], out[1, 1:_NN, :]], axis=-1)
    return res.T


# speedup vs baseline: 20.9939x; 2.0493x over previous
import functools

import jax
import jax.numpy as jnp
from jax import lax
from jax.experimental import pallas as pl
from jax.experimental.pallas import tpu as pltpu
from jax.experimental.pallas import tpu_sc as plsc

_N = 100000
_NN = _N + 1
_B = 32
_E = 1600000
_NC = 2
_NS = 16
_L = 16

_EPAD = 1638400
_ROWS = _EPAD // 128
_RPW = _ROWS // _NS
_CR = 1
_CHUNKS = _RPW // _CR
_CE = _CR * 128
_NB = 4

_WPN = 6256
_NP = _WPN * _NS

_mesh = plsc.VectorSubcoreMesh(core_axis_name="c", subcore_axis_name="s")

_set = lambda: [
    pltpu.VMEM((_CR, 128), jnp.int32),
    pltpu.VMEM((_CR, 128), jnp.int32),
    pltpu.VMEM((_CR, 128), jnp.float32),
    pltpu.VMEM((_CE, _L), jnp.float32),
    pltpu.VMEM((_CE, _L), jnp.float32),
]

_SEMS = [pltpu.SemaphoreType.DMA] * (3 * _NB)


@jax.jit
def _circuit_sc(xtr, src2, des2, par2):
    @functools.partial(
        pl.kernel,
        out_type=jax.ShapeDtypeStruct((_NC, _NP, _L), jnp.float32),
        mesh=_mesh,
        scratch_types=sum([_set() for _ in range(_NB)], []) + [
            pltpu.VMEM_SHARED((_NP, _L), jnp.float32),
        ] + _SEMS,
        compiler_params=pltpu.CompilerParams(use_tc_tiling_on_sc=False),
    )
    def k(xtr_hbm, src_hbm, des_hbm, par_hbm, out_hbm, *rest):
        bufs = []
        for p in range(_NB):
            bufs.append(tuple(rest[5 * p: 5 * p + 5]))
        acc = rest[5 * _NB]
        sems = rest[5 * _NB + 1:]
        sets = tuple(bufs[p] + (sems[3 * p], sems[3 * p + 1], sems[3 * p + 2])
                     for p in range(_NB))

        c = lax.axis_index("c")
        s = lax.axis_index("s")
        vs0 = sets[0][3]

        @pl.loop(0, _CE, unroll=8)
        def _(i):
            vs0[i] = jnp.zeros((_L,), jnp.float32)

        zbase = s * _WPN
        for tk in range(_WPN // _CE):
            pltpu.sync_copy(vs0.at[pl.ds(0, _CE)],
                            acc.at[pl.ds(zbase + tk * _CE, _CE)])
        _rem = _WPN % _CE
        if _rem:
            pltpu.sync_copy(vs0.at[pl.ds(0, _rem)],
                            acc.at[pl.ds(zbase + (_WPN // _CE) * _CE, _rem)])
        plsc.subcore_barrier()

        row_base = s * _RPW
        table = xtr_hbm.at[c]

        def fire_loads(chunk, p):
            (srcb, desb, parb, _, _, lsem, _, _) = sets[p]
            off = row_base + chunk * _CR
            pltpu.async_copy(src_hbm.at[pl.ds(off, _CR)], srcb, lsem)
            pltpu.async_copy(des_hbm.at[pl.ds(off, _CR)], desb, lsem)
            pltpu.async_copy(par_hbm.at[pl.ds(off, _CR)], parb, lsem)

        def drain_loads(p):
            (srcb, desb, parb, _, _, lsem, _, _) = sets[p]
            pltpu.make_async_copy(src_hbm.at[pl.ds(0, _CR)], srcb, lsem).wait()
            pltpu.make_async_copy(des_hbm.at[pl.ds(0, _CR)], desb, lsem).wait()
            pltpu.make_async_copy(par_hbm.at[pl.ds(0, _CR)], parb, lsem).wait()

        def fire_gathers(p):
            (srcb, desb, _, vsb, vdb, _, gsem, _) = sets[p]
            for j in range(_CR):
                pltpu.async_copy(table.at[srcb.at[j]],
                                 vsb.at[pl.ds(j * 128, 128)], gsem)
                pltpu.async_copy(table.at[desb.at[j]],
                                 vdb.at[pl.ds(j * 128, 128)], gsem)

        def drain_gathers(p):
            (srcb, desb, _, vsb, vdb, _, gsem, _) = sets[p]
            for j in range(_CR):
                pltpu.make_async_copy(table.at[srcb.at[j]],
                                      vsb.at[pl.ds(j * 128, 128)], gsem).wait()
                pltpu.make_async_copy(table.at[desb.at[j]],
                                      vdb.at[pl.ds(j * 128, 128)], gsem).wait()

        def compute_and_fire_scatters(p):
            (srcb, desb, parb, vsb, vdb, _, _, ssem) = sets[p]
            for r in range(_CR):
                @plsc.parallel_loop(0, 128, 16)
                def _(e0, r=r):
                    pv = parb[r, pl.ds(e0, _L)]
                    for i in range(_L):
                        e = r * 128 + e0 + i
                        row = pv[i] * (vsb[e] - vdb[e])
                        vsb[e] = row
                        vdb[e] = -row
            for j in range(_CR):
                pltpu.async_copy(vdb.at[pl.ds(j * 128, 128)],
                                 acc.at[srcb.at[j]], ssem, add=True)
                pltpu.async_copy(vsb.at[pl.ds(j * 128, 128)],
                                 acc.at[desb.at[j]], ssem, add=True)

        def drain_scatters(p):
            (srcb, desb, _, vsb, vdb, _, _, ssem) = sets[p]
            for j in range(_CR):
                pltpu.make_async_copy(vdb.at[pl.ds(j * 128, 128)],
                                      acc.at[srcb.at[j]], ssem).wait()
                pltpu.make_async_copy(vsb.at[pl.ds(j * 128, 128)],
                                      acc.at[desb.at[j]], ssem).wait()

        def phase(p, k_prev_next, k_gather, *, drain_sc=True,
                  fire_ld=True, fire_ga=True):
            prev = (p + _NB - 1) % _NB
            q2 = (p + _NB - 2) % _NB
            if drain_sc:
                drain_scatters(prev)
            if fire_ld:
                fire_loads(k_prev_next, prev)
            if fire_ga:
                drain_loads(q2)
                fire_gathers(q2)
            drain_gathers(p)
            compute_and_fire_scatters(p)

        for p in range(_NB - 1):
            fire_loads(p, p)
        for p in range(2):
            drain_loads(p)
            fire_gathers(p)

        phase(0, _NB - 1, 0, drain_sc=False)
        for p in range(1, _NB):
            phase(p, p - 1 + _NB, p)

        @pl.loop(_NB, _CHUNKS - _NB, step=_NB)
        def _(base):
            for p in range(_NB):
                phase(p, base + p - 1 + _NB, base + p)

        phase(0, _CHUNKS - 1, _CHUNKS - _NB)
        phase(1, 0, _CHUNKS - _NB + 1, fire_ld=False)
        for p in range(2, _NB):
            phase(p, 0, 0, fire_ld=False, fire_ga=False)
        drain_scatters(_NB - 1)

        plsc.subcore_barrier()

        wbase = s * _WPN
        for tk in range(_WPN // _CE):
            pltpu.sync_copy(acc.at[pl.ds(wbase + tk * _CE, _CE)],
                            out_hbm.at[c, pl.ds(wbase + tk * _CE, _CE)])
        if _rem:
            pltpu.sync_copy(acc.at[pl.ds(wbase + (_WPN // _CE) * _CE, _rem)],
                            out_hbm.at[c, pl.ds(wbase + (_WPN // _CE) * _CE, _rem)])

    return k(xtr, src2, des2, par2)


def kernel(t, x, src, des, param):
    del t
    aux_t = jnp.concatenate([jnp.zeros((1, _B), x.dtype), x.T], axis=0)
    xtr = aux_t.reshape(_NN, _NC, _L).transpose(1, 0, 2)

    pad = _EPAD - _E
    pad_idx = (jnp.arange(pad, dtype=jnp.int32) % _N) + 1
    src2 = jnp.concatenate([src, pad_idx]).reshape(_ROWS, 128)
    des2 = jnp.concatenate([des, pad_idx]).reshape(_ROWS, 128)
    par2 = jnp.concatenate([param, jnp.zeros((pad,), param.dtype)]).reshape(_ROWS, 128)

    out = _circuit_sc(xtr, src2, des2, par2)
    res = jnp.concatenate([out[0, 1:_NN, :], out[1, 1:_NN, :]], axis=-1)
    return res.T
